# Initial kernel scaffold; baseline (speedup 1.0000x reference)
import jax
import jax.numpy as jnp
from jax.experimental import pallas as pl


def kernel(hidden_state, input_ids, sparse_W, sparse_b):
    raise NotImplementedError("write your pallas kernel here")


# speedup vs baseline: 2.1971x; 2.1971x over previous
import functools

import jax
import jax.numpy as jnp
from jax import lax
from jax.experimental import pallas as pl
from jax.experimental.pallas import tpu as pltpu
from jax.experimental.pallas import tpu_sc as plsc

_B, _L, _H, _V = 16, 512, 1024, 250002
_NC, _NS = 2, 16
_NW = _NC * _NS
_REGION = 125008
_REGION_LAST = _B * _V - (_NW - 1) * _REGION
_CHUNK = 9616
_NCH = _REGION // _CHUNK
_CHUNK_LAST = _REGION_LAST - (_NCH - 1) * _CHUNK


def _tc_body(w_ref, b_ref, hs_ref, ids_ref, ids3_ref, out_ref):
    hs = hs_ref[0]
    w = w_ref[...]
    bias = b_ref[0]
    w128 = jnp.broadcast_to(w, (128, _H))
    prod = lax.dot_general(hs, w128, (((1,), (1,)), ((), ())),
                           preferred_element_type=jnp.float32)
    tw_col = jnp.maximum(prod[:, 0:1] + bias, 0.0)
    ids_row = ids_ref[0]
    ids_col = ids3_ref[0]
    for c in range(_L // 64):
        idr = ids_row[:, c * 64:(c + 1) * 64]
        eq = ids_col == idr
        contrib = jnp.where(eq, tw_col, 0.0)
        m = jnp.max(contrib, axis=0, keepdims=True)
        m = jnp.where(idr < 4, 0.0, m)
        out_ref[0, :, c * 64:(c + 1) * 64] = m


def _sc_body(ids_hbm, vals_hbm, out_hbm, region_v, ids_v, vals_v, locals_v,
             sem):
    c = lax.axis_index("c")
    s = lax.axis_index("s")
    wid = s * _NC + c
    lo_flat = wid * _REGION
    span = jnp.where(wid == _NW - 1, _REGION_LAST, _REGION)

    b0 = wid // 2
    b1 = jnp.minimum((wid + 1) // 2, _B - 1)

    pltpu.sync_copy(ids_hbm.at[pl.ds(pl.multiple_of(b0 * _L, 8), _L)],
                    ids_v.at[pl.ds(0, _L)])
    pltpu.sync_copy(ids_hbm.at[pl.ds(pl.multiple_of(b1 * _L, 8), _L)],
                    ids_v.at[pl.ds(_L, _L)])
    pltpu.sync_copy(vals_hbm.at[pl.ds(pl.multiple_of(b0 * _L, 8), _L)],
                    vals_v.at[pl.ds(0, _L)])
    pltpu.sync_copy(vals_hbm.at[pl.ds(pl.multiple_of(b1 * _L, 8), _L)],
                    vals_v.at[pl.ds(_L, _L)])

    def _loc(i, carry):
        bi = jnp.where(i < _L // 16, b0, b1)
        idv = ids_v[pl.ds(pl.multiple_of(i * 16, 16), 16)]
        fl = bi * _V + idv - lo_flat
        ok = (fl >= 0) & (fl < span)
        locals_v[pl.ds(pl.multiple_of(i * 16, 16), 16)] = jnp.where(
            ok, fl, jnp.int32(1 << 30))
        return carry
    lax.fori_loop(0, 2 * _L // 16, _loc, 0)

    def _zero(ck):
        base = ck * _CHUNK

        def zbody(j, carry):
            region_v[pl.ds(pl.multiple_of(base + j * 16, 16), 16)] = (
                jnp.zeros((16,), jnp.float32))
            return carry
        lax.fori_loop(0, _CHUNK // 16, zbody, 0)

    def _scatter(ck):
        base = ck * _CHUNK

        def sbody(i, carry):
            loc = locals_v[pl.ds(pl.multiple_of(i * 16, 16), 16)]
            rel = loc - base
            m = (rel >= 0) & (rel < _CHUNK)
            v = vals_v[pl.ds(pl.multiple_of(i * 16, 16), 16)]
            plsc.store_scatter(region_v, [jnp.where(m, loc, 0)], v, mask=m)
            return carry
        lax.fori_loop(0, 2 * _L // 16, sbody, 0)

    copies = []
    for ck in range(_NCH - 1):
        _zero(ck)
        _scatter(ck)
        copies.append(pltpu.async_copy(
            region_v.at[pl.ds(ck * _CHUNK, _CHUNK)],
            out_hbm.at[pl.ds(pl.multiple_of(lo_flat + ck * _CHUNK, 8),
                             _CHUNK)],
            sem))

    ck = _NCH - 1
    _zero(ck)
    _scatter(ck)

    @pl.when(wid != _NW - 1)
    def _last_full():
        pltpu.async_copy(
            region_v.at[pl.ds(ck * _CHUNK, _CHUNK)],
            out_hbm.at[pl.ds(pl.multiple_of(lo_flat + ck * _CHUNK, 8),
                             _CHUNK)],
            sem).wait()

    @pl.when(wid == _NW - 1)
    def _last_short():
        pltpu.async_copy(
            region_v.at[pl.ds(ck * _CHUNK, _CHUNK_LAST)],
            out_hbm.at[pl.ds(pl.multiple_of(lo_flat + ck * _CHUNK, 8),
                             _CHUNK_LAST)],
            sem).wait()

    for cp in copies:
        cp.wait()


@functools.lru_cache(maxsize=1)
def _sc_scatter_fn():
    return functools.partial(
        pl.kernel,
        out_type=jax.ShapeDtypeStruct((_B * _V,), jnp.float32),
        mesh=plsc.VectorSubcoreMesh(
            core_axis_name="c", subcore_axis_name="s",
            num_cores=_NC, num_subcores=_NS),
        compiler_params=pltpu.CompilerParams(needs_layout_passes=False),
        scratch_types=[
            pltpu.VMEM((_REGION,), jnp.float32),
            pltpu.VMEM((2 * _L,), jnp.int32),
            pltpu.VMEM((2 * _L,), jnp.float32),
            pltpu.VMEM((2 * _L,), jnp.int32),
            pltpu.SemaphoreType.DMA,
        ],
    )(_sc_body)


def kernel(hidden_state, input_ids, sparse_W, sparse_b):
    ids2 = input_ids.reshape(_B, 1, _L)
    ids3 = input_ids.reshape(_B, _L, 1)
    combined = pl.pallas_call(
        _tc_body,
        grid=(_B,),
        in_specs=[
            pl.BlockSpec((1, _H), lambda b: (0, 0)),
            pl.BlockSpec(memory_space=pltpu.SMEM),
            pl.BlockSpec((1, _L, _H), lambda b: (b, 0, 0)),
            pl.BlockSpec((1, 1, _L), lambda b: (b, 0, 0)),
            pl.BlockSpec((1, _L, 1), lambda b: (b, 0, 0)),
        ],
        out_specs=pl.BlockSpec((1, 1, _L), lambda b: (b, 0, 0)),
        out_shape=jax.ShapeDtypeStruct((_B, 1, _L), jnp.float32),
    )(sparse_W, sparse_b, hidden_state, ids2, ids3)
    flat = _sc_scatter_fn()(input_ids.reshape(-1), combined.reshape(-1))
    return flat.reshape(_B, _V)


# speedup vs baseline: 2.9055x; 1.3224x over previous
import functools

import jax
import jax.numpy as jnp
from jax import lax
from jax.experimental import pallas as pl
from jax.experimental.pallas import tpu as pltpu
from jax.experimental.pallas import tpu_sc as plsc

_B, _L, _H, _V = 16, 512, 1024, 250002
_NC, _NS = 2, 16
_NW = _NC * _NS
_REGION = 125008
_REGION_LAST = _B * _V - (_NW - 1) * _REGION
_CHUNK = 9616
_NCH = _REGION // _CHUNK
_CHUNK_LAST = _REGION_LAST - (_NCH - 1) * _CHUNK


def _tc_body(w_ref, b_ref, hs_ref, ids_ref, ids3_ref, out_ref):
    hs = hs_ref[0]
    w = w_ref[...]
    bias = b_ref[0]
    w128 = jnp.broadcast_to(w, (128, _H))
    prod = lax.dot_general(hs, w128, (((1,), (1,)), ((), ())),
                           preferred_element_type=jnp.float32)
    tw_col = jnp.maximum(prod[:, 0:1] + bias, 0.0)
    ids_row = ids_ref[0]
    ids_col = ids3_ref[0]
    for c in range(_L // 64):
        idr = ids_row[:, c * 64:(c + 1) * 64]
        eq = ids_col == idr
        contrib = jnp.where(eq, tw_col, 0.0)
        m = jnp.max(contrib, axis=0, keepdims=True)
        m = jnp.where(idr < 4, 0.0, m)
        out_ref[0, :, c * 64:(c + 1) * 64] = m


def _sc_body(ids_hbm, vals_hbm, out_hbm, region_v, ids_v, vals_v, locals_v,
             sem):
    c = lax.axis_index("c")
    s = lax.axis_index("s")
    wid = s * _NC + c
    lo_flat = wid * _REGION
    span = jnp.where(wid == _NW - 1, _REGION_LAST, _REGION)

    b0 = wid // 2
    b1 = jnp.minimum((wid + 1) // 2, _B - 1)

    pltpu.sync_copy(ids_hbm.at[pl.ds(pl.multiple_of(b0 * _L, 8), _L)],
                    ids_v.at[pl.ds(0, _L)])
    pltpu.sync_copy(ids_hbm.at[pl.ds(pl.multiple_of(b1 * _L, 8), _L)],
                    ids_v.at[pl.ds(_L, _L)])
    pltpu.sync_copy(vals_hbm.at[pl.ds(pl.multiple_of(b0 * _L, 8), _L)],
                    vals_v.at[pl.ds(0, _L)])
    pltpu.sync_copy(vals_hbm.at[pl.ds(pl.multiple_of(b1 * _L, 8), _L)],
                    vals_v.at[pl.ds(_L, _L)])

    def _loc(i, carry):
        bi = jnp.where(i < _L // 16, b0, b1)
        idv = ids_v[pl.ds(pl.multiple_of(i * 16, 16), 16)]
        fl = bi * _V + idv - lo_flat
        ok = (fl >= 0) & (fl < span)
        locals_v[pl.ds(pl.multiple_of(i * 16, 16), 16)] = jnp.where(
            ok, fl, jnp.int32(1 << 30))
        return carry
    lax.fori_loop(0, 2 * _L // 16, _loc, 0)

    zvec = jnp.zeros((16,), jnp.float32)

    def _zero(ck):
        base = ck * _CHUNK

        def zbody(j, carry):
            off = pl.multiple_of(base + j * 128, 16)
            for u in range(8):
                region_v[pl.ds(off + u * 16, 16)] = zvec
            return carry
        lax.fori_loop(0, _CHUNK // 128, zbody, 0)
        region_v[pl.ds(base + _CHUNK - 16, 16)] = zvec

    def _scatter(ck):
        base = ck * _CHUNK

        def sbody(i, carry):
            off = pl.multiple_of(i * 64, 16)
            for u in range(4):
                loc = locals_v[pl.ds(off + u * 16, 16)]
                rel = loc - base
                m = (rel >= 0) & (rel < _CHUNK)
                v = vals_v[pl.ds(off + u * 16, 16)]
                plsc.store_scatter(region_v, [jnp.where(m, loc, 0)], v,
                                   mask=m)
            return carry
        lax.fori_loop(0, 2 * _L // 64, sbody, 0)

    copies = []
    for ck in range(_NCH - 1):
        _zero(ck)
        _scatter(ck)
        copies.append(pltpu.async_copy(
            region_v.at[pl.ds(ck * _CHUNK, _CHUNK)],
            out_hbm.at[pl.ds(pl.multiple_of(lo_flat + ck * _CHUNK, 8),
                             _CHUNK)],
            sem))

    ck = _NCH - 1
    _zero(ck)
    _scatter(ck)

    @pl.when(wid != _NW - 1)
    def _last_full():
        pltpu.async_copy(
            region_v.at[pl.ds(ck * _CHUNK, _CHUNK)],
            out_hbm.at[pl.ds(pl.multiple_of(lo_flat + ck * _CHUNK, 8),
                             _CHUNK)],
            sem).wait()

    @pl.when(wid == _NW - 1)
    def _last_short():
        pltpu.async_copy(
            region_v.at[pl.ds(ck * _CHUNK, _CHUNK_LAST)],
            out_hbm.at[pl.ds(pl.multiple_of(lo_flat + ck * _CHUNK, 8),
                             _CHUNK_LAST)],
            sem).wait()

    for cp in copies:
        cp.wait()


@functools.lru_cache(maxsize=1)
def _sc_scatter_fn():
    return functools.partial(
        pl.kernel,
        out_type=jax.ShapeDtypeStruct((_B * _V,), jnp.float32),
        mesh=plsc.VectorSubcoreMesh(
            core_axis_name="c", subcore_axis_name="s",
            num_cores=_NC, num_subcores=_NS),
        compiler_params=pltpu.CompilerParams(needs_layout_passes=False),
        scratch_types=[
            pltpu.VMEM((_REGION,), jnp.float32),
            pltpu.VMEM((2 * _L,), jnp.int32),
            pltpu.VMEM((2 * _L,), jnp.float32),
            pltpu.VMEM((2 * _L,), jnp.int32),
            pltpu.SemaphoreType.DMA,
        ],
    )(_sc_body)


def kernel(hidden_state, input_ids, sparse_W, sparse_b):
    ids2 = input_ids.reshape(_B, 1, _L)
    ids3 = input_ids.reshape(_B, _L, 1)
    combined = pl.pallas_call(
        _tc_body,
        grid=(_B,),
        in_specs=[
            pl.BlockSpec((1, _H), lambda b: (0, 0)),
            pl.BlockSpec(memory_space=pltpu.SMEM),
            pl.BlockSpec((1, _L, _H), lambda b: (b, 0, 0)),
            pl.BlockSpec((1, 1, _L), lambda b: (b, 0, 0)),
            pl.BlockSpec((1, _L, 1), lambda b: (b, 0, 0)),
        ],
        out_specs=pl.BlockSpec((1, 1, _L), lambda b: (b, 0, 0)),
        out_shape=jax.ShapeDtypeStruct((_B, 1, _L), jnp.float32),
    )(sparse_W, sparse_b, hidden_state, ids2, ids3)
    flat = _sc_scatter_fn()(input_ids.reshape(-1), combined.reshape(-1))
    return flat.reshape(_B, _V)
